# baseline (speedup 1.0000x reference)
import jax
import jax.numpy as jnp
from jax import lax
from jax.experimental import pallas as pl
from jax.experimental.pallas import tpu as pltpu
from jax.experimental.pallas import tpu_sc as plsc

N = 10000
D = 128
E = 320000
NEG_SLOPE = 0.2
NC, NS, L = 2, 16, 16
NW = NC * NS
EPW = E // NW
CHUNK = 80
NCHUNK = EPW // CHUNK
COVER = 640
LASTN0 = N - COVER


def _mm_body(x_ref, w_ref, a_ref, h1_ref, h2_ref, s1_ref, s2_ref):
    x = x_ref[...]
    w = w_ref[...]
    av = a_ref[...]
    dn = (((1,), (1,)), ((), ()))
    h1 = lax.dot_general(x, w[:, :D], dn, preferred_element_type=jnp.float32)
    h2 = lax.dot_general(x, w[:, D:], dn, preferred_element_type=jnp.float32)
    h1_ref[...] = h1
    h2_ref[...] = h2
    s1_ref[...] = lax.dot_general(h1, av, dn, preferred_element_type=jnp.float32)
    s2_ref[...] = lax.dot_general(h2, av, dn, preferred_element_type=jnp.float32)


_mm_call = pl.pallas_call(
    _mm_body,
    out_shape=[
        jax.ShapeDtypeStruct((N, D), jnp.float32),
        jax.ShapeDtypeStruct((N, D), jnp.float32),
        jax.ShapeDtypeStruct((N, 1), jnp.float32),
        jax.ShapeDtypeStruct((N, 1), jnp.float32),
    ],
)


def _sc_body(epk_hbm, s1_hbm, s2_hbm, h2_hbm, part_hbm, rsum_hbm,
             s1_v, s2_v, pk_v, e0i_v, e1i_v, wv_v, rowsa_v, rowsb_v,
             acc_sh, rs_sh, semi, semg, sems):
    cid = lax.axis_index("c")
    sid = lax.axis_index("s")
    wid = cid * NS + sid
    base = wid * EPW

    rows_bufs = (rowsa_v, rowsb_v)

    pltpu.sync_copy(s1_hbm, s1_v)
    pltpu.sync_copy(s2_hbm, s2_v)

    zrow0 = jnp.zeros((L,), jnp.float32)

    def zrow(r, carry):
        for j in range(D // L):
            rowsa_v[r, pl.ds(j * L, L)] = zrow0
            rowsb_v[r, pl.ds(j * L, L)] = zrow0
        return carry

    lax.fori_loop(0, CHUNK, zrow, 0)
    for b in range(2):
        for j in range(CHUNK // L):
            wv_v[b, pl.ds(j * L, L)] = zrow0

    n0 = jnp.minimum(sid * COVER, LASTN0)
    for k in range(COVER // CHUNK):
        pltpu.sync_copy(rowsa_v, acc_sh.at[pl.ds(n0 + k * CHUNK, CHUNK)])
        pltpu.sync_copy(wv_v.at[0], rs_sh.at[pl.ds(n0 + k * CHUNK, CHUNK)])
    plsc.subcore_barrier()

    def i_issue(c, ib):
        pltpu.async_copy(epk_hbm.at[pl.ds(base + c * CHUNK, CHUNK)],
                         pk_v.at[ib], semi[ib])

    def i_wait(ib):
        pltpu.make_async_copy(epk_hbm.at[pl.ds(0, CHUNK)],
                              pk_v.at[ib], semi[ib]).wait()

    def idx_unpack(ib, rb):
        for g in range(CHUNK // L):
            pk = pk_v[ib, pl.ds(g * L, L)]
            e0i_v[rb, pl.ds(g * L, L)] = pk & 0xFFFF
            e1i_v[rb, pl.ds(g * L, L)] = lax.shift_right_logical(pk, 16)

    H = CHUNK // 2

    def g_issue(rb):
        pltpu.async_copy(h2_hbm.at[e1i_v.at[rb, pl.ds(0, H)]],
                         rows_bufs[rb].at[pl.ds(0, H)], semg[rb])
        pltpu.async_copy(h2_hbm.at[e1i_v.at[rb, pl.ds(H, H)]],
                         rows_bufs[rb].at[pl.ds(H, H)], semg[rb])

    def g_wait(rb):
        pltpu.make_async_copy(h2_hbm.at[pl.ds(0, H)],
                              rows_bufs[rb].at[pl.ds(0, H)], semg[rb]).wait()
        pltpu.make_async_copy(h2_hbm.at[pl.ds(0, H)],
                              rows_bufs[rb].at[pl.ds(H, H)], semg[rb]).wait()

    def s_issue(rb):
        pltpu.async_copy(rows_bufs[rb], acc_sh.at[e0i_v.at[rb]],
                         sems[rb], add=True)
        pltpu.async_copy(wv_v.at[rb], rs_sh.at[e0i_v.at[rb]],
                         sems[rb], add=True)

    def s_wait(rb):
        pltpu.make_async_copy(rows_bufs[rb], acc_sh.at[pl.ds(0, CHUNK)],
                              sems[rb]).wait()
        pltpu.make_async_copy(wv_v.at[rb], rs_sh.at[pl.ds(0, CHUNK)],
                              sems[rb]).wait()

    def compute_scale(rb):
        rows = rows_bufs[rb]
        for g in range(CHUNK // L):
            i0 = e0i_v[rb, pl.ds(g * L, L)]
            i1 = e1i_v[rb, pl.ds(g * L, L)]
            x = plsc.load_gather(s1_v, [i0]) + plsc.load_gather(s2_v, [i1])
            wv_v[rb, pl.ds(g * L, L)] = jnp.exp(-jnp.maximum(x, NEG_SLOPE * x))

        def srow(g, carry2):
            wgrp = wv_v[rb, pl.ds(g * L, L)]
            for u in range(L):
                i = g * L + u
                wv = wgrp[u]
                for j in range(D // L):
                    rows[i, pl.ds(j * L, L)] = rows[i, pl.ds(j * L, L)] * wv
            return carry2

        lax.fori_loop(0, CHUNK // L, srow, 0)

    i_issue(0, 0)
    i_issue(1, 1)
    i_wait(0)
    idx_unpack(0, 0)
    g_issue(0)
    pltpu.async_copy(rowsb_v, acc_sh.at[e0i_v.at[0]], sems[1], add=True)
    pltpu.async_copy(wv_v.at[1], rs_sh.at[e0i_v.at[0]], sems[1], add=True)

    def step(c, off):
        rb = off % 2
        ib = off % 4
        g_wait(rb)
        s_wait((rb + 1) % 2)
        i_wait((ib + 1) % 4)
        idx_unpack((ib + 1) % 4, (rb + 1) % 2)
        g_issue((rb + 1) % 2)

        @pl.when(c + 2 < NCHUNK)
        def _():
            i_issue(c + 2, (ib + 2) % 4)

        compute_scale(rb)
        s_issue(rb)

    def quad(it, carry):
        c = 4 * it
        for off in range(4):
            step(c + off, off)
        return carry

    lax.fori_loop(0, NCHUNK // 4, quad, 0)
    g_wait(0)
    s_wait(1)
    compute_scale(0)
    s_issue(0)
    s_wait(0)
    plsc.subcore_barrier()

    pltpu.sync_copy(acc_sh.at[pl.ds(n0, COVER)],
                    part_hbm.at[cid, pl.ds(n0, COVER)])
    pltpu.sync_copy(rs_sh.at[pl.ds(n0, COVER)],
                    rsum_hbm.at[cid, pl.ds(n0, COVER)])


_sc_call = pl.kernel(
    _sc_body,
    out_type=(
        jax.ShapeDtypeStruct((NC, N, D), jnp.float32),
        jax.ShapeDtypeStruct((NC, N), jnp.float32),
    ),
    mesh=plsc.VectorSubcoreMesh(core_axis_name="c", subcore_axis_name="s",
                                num_cores=NC, num_subcores=NS),
    compiler_params=pltpu.CompilerParams(use_tc_tiling_on_sc=False,
                                         needs_layout_passes=False),
    scratch_types=[
        pltpu.VMEM((N,), jnp.float32),
        pltpu.VMEM((N,), jnp.float32),
        pltpu.VMEM((4, CHUNK), jnp.int32),
        pltpu.VMEM((2, CHUNK), jnp.int32),
        pltpu.VMEM((2, CHUNK), jnp.int32),
        pltpu.VMEM((2, CHUNK), jnp.float32),
        pltpu.VMEM((CHUNK, D), jnp.float32),
        pltpu.VMEM((CHUNK, D), jnp.float32),
        pltpu.VMEM_SHARED((N, D), jnp.float32),
        pltpu.VMEM_SHARED((N,), jnp.float32),
        [pltpu.SemaphoreType.DMA] * 4,
        [pltpu.SemaphoreType.DMA] * 2,
        [pltpu.SemaphoreType.DMA] * 2,
    ],
)


def _comb_body(h1_ref, part_ref, rsum_ref, o_ref):
    acc = part_ref[0] + part_ref[1]
    rs = rsum_ref[0] + rsum_ref[1]
    denom = jnp.where(rs == 0.0, 1e-12, rs)
    h = (h1_ref[...] * rs + acc) / denom
    o_ref[...] = jnp.where(h > 0.0, h, jnp.exp(h) - 1.0)


_comb_call = pl.pallas_call(
    _comb_body,
    out_shape=jax.ShapeDtypeStruct((N, D), jnp.float32),
)


def kernel(input_, edge, W, a):
    edge = edge.astype(jnp.int32)
    epk = edge[0] | (edge[1] << 16)
    h1, h2, s1, s2 = _mm_call(input_, W, a)
    part, rsum = _sc_call(epk, s1.reshape(N), s2.reshape(N), h2)
    return _comb_call(h1, part, rsum.reshape(NC, N, 1))


# speedup vs baseline: 1.0693x; 1.0693x over previous
import jax
import jax.numpy as jnp
from jax import lax
from jax.experimental import pallas as pl
from jax.experimental.pallas import tpu as pltpu
from jax.experimental.pallas import tpu_sc as plsc

N = 10000
D = 128
E = 320000
NEG_SLOPE = 0.2
NC, NS, L = 2, 16, 16
NW = NC * NS
EPW = E // NW
CHUNK = 80
NCHUNK = EPW // CHUNK
COVER = 640
LASTN0 = N - COVER


def _mm_body(x_ref, w_ref, a_ref, h1_ref, h2_ref, s1_ref, s2_ref):
    x = x_ref[...]
    w = w_ref[...]
    av = a_ref[...]
    dn = (((1,), (1,)), ((), ()))
    h1 = lax.dot_general(x, w[:, :D], dn, preferred_element_type=jnp.float32)
    h2 = lax.dot_general(x, w[:, D:], dn, preferred_element_type=jnp.float32)
    h1_ref[...] = h1
    h2_ref[...] = h2
    s1_ref[...] = lax.dot_general(h1, av, dn, preferred_element_type=jnp.float32)
    s2_ref[...] = lax.dot_general(h2, av, dn, preferred_element_type=jnp.float32)


_mm_call = pl.pallas_call(
    _mm_body,
    out_shape=[
        jax.ShapeDtypeStruct((N, D), jnp.float32),
        jax.ShapeDtypeStruct((N, D), jnp.float32),
        jax.ShapeDtypeStruct((N, 1), jnp.float32),
        jax.ShapeDtypeStruct((N, 1), jnp.float32),
    ],
)


def _sc_body(epk_hbm, st_hbm, h2_hbm, part_hbm, rsum_hbm,
             st_v, pk_v, e0i_v, e1i_v, wv_v, rowsa_v, rowsb_v, rowsc_v,
             acc_sh, rs_sh, semi, semg, sems):
    cid = lax.axis_index("c")
    sid = lax.axis_index("s")
    wid = cid * NS + sid
    base = wid * EPW

    rows_bufs = (rowsa_v, rowsb_v, rowsc_v)

    pltpu.sync_copy(st_hbm, st_v)

    zrow0 = jnp.zeros((L,), jnp.float32)

    def zrow(r, carry):
        for j in range(D // L):
            rowsa_v[r, pl.ds(j * L, L)] = zrow0
            rowsb_v[r, pl.ds(j * L, L)] = zrow0
            rowsc_v[r, pl.ds(j * L, L)] = zrow0
        return carry

    lax.fori_loop(0, CHUNK, zrow, 0)
    for b in range(3):
        for j in range(CHUNK // L):
            wv_v[b, pl.ds(j * L, L)] = zrow0

    n0 = jnp.minimum(sid * COVER, LASTN0)
    for k in range(COVER // CHUNK):
        pltpu.sync_copy(rowsa_v, acc_sh.at[pl.ds(n0 + k * CHUNK, CHUNK)])
        pltpu.sync_copy(wv_v.at[0], rs_sh.at[pl.ds(n0 + k * CHUNK, CHUNK)])
    plsc.subcore_barrier()

    def i_issue(c, ib):
        pltpu.async_copy(epk_hbm.at[pl.ds(base + c * CHUNK, CHUNK)],
                         pk_v.at[ib], semi[ib])

    def i_wait(ib):
        pltpu.make_async_copy(epk_hbm.at[pl.ds(0, CHUNK)],
                              pk_v.at[ib], semi[ib]).wait()

    def idx_unpack(ib, rb):
        for g in range(CHUNK // L):
            pk = pk_v[ib, pl.ds(g * L, L)]
            e0i_v[rb, pl.ds(g * L, L)] = pk & 0xFFFF
            e1i_v[rb, pl.ds(g * L, L)] = lax.shift_right_logical(pk, 16)

    def g_issue(rb):
        pltpu.async_copy(h2_hbm.at[e1i_v.at[rb]], rows_bufs[rb], semg[rb])

    def g_wait(rb):
        pltpu.make_async_copy(h2_hbm.at[pl.ds(0, CHUNK)],
                              rows_bufs[rb], semg[rb]).wait()

    def s_issue(rb):
        pltpu.async_copy(rows_bufs[rb], acc_sh.at[e0i_v.at[rb]],
                         sems[rb], add=True)
        pltpu.async_copy(wv_v.at[rb], rs_sh.at[e0i_v.at[rb]],
                         sems[rb], add=True)

    def s_wait(rb):
        pltpu.make_async_copy(rows_bufs[rb], acc_sh.at[pl.ds(0, CHUNK)],
                              sems[rb]).wait()
        pltpu.make_async_copy(wv_v.at[rb], rs_sh.at[pl.ds(0, CHUNK)],
                              sems[rb]).wait()

    def compute_scale(rb):
        rows = rows_bufs[rb]
        for g in range(CHUNK // L):
            i0 = e0i_v[rb, pl.ds(g * L, L)]
            i1 = e1i_v[rb, pl.ds(g * L, L)]
            ga = plsc.load_gather(st_v, [i0])
            gb = plsc.load_gather(st_v, [i1])
            x = (plsc.bitcast(ga << 16, jnp.float32)
                 + plsc.bitcast(gb & (-65536), jnp.float32))
            wv_v[rb, pl.ds(g * L, L)] = jnp.exp(-jnp.maximum(x, NEG_SLOPE * x))

        def srow(g, carry2):
            wgrp = wv_v[rb, pl.ds(g * L, L)]
            for u in range(L):
                i = g * L + u
                wv = wgrp[u]
                for j in range(D // L):
                    rows[i, pl.ds(j * L, L)] = rows[i, pl.ds(j * L, L)] * wv
            return carry2

        lax.fori_loop(0, CHUNK // L, srow, 0)

    i_issue(0, 0)
    i_issue(1, 1)
    i_issue(2, 2)
    i_wait(0)
    idx_unpack(0, 0)
    i_wait(1)
    idx_unpack(1, 1)
    g_issue(0)
    g_issue(1)
    pltpu.async_copy(rowsc_v, acc_sh.at[e0i_v.at[0]], sems[2], add=True)
    pltpu.async_copy(wv_v.at[2], rs_sh.at[e0i_v.at[0]], sems[2], add=True)

    def step(c, off, live_i=True, live_g=True, live_w=True):
        rb = off % 3
        ib = off % 4
        g_wait(rb)
        s_wait((rb + 2) % 3)
        if live_w:
            i_wait((ib + 2) % 4)
            idx_unpack((ib + 2) % 4, (rb + 2) % 3)
        if live_g:
            g_issue((rb + 2) % 3)
        if live_i:
            i_issue(c + 3, (ib + 3) % 4)
        compute_scale(rb)
        s_issue(rb)

    def twelve(it, carry):
        c = 12 * it
        for off in range(12):
            step(c + off, off)
        return carry

    lax.fori_loop(0, NCHUNK // 12, twelve, 0)
    step(120, 120, live_i=(123 < NCHUNK), live_g=True, live_w=True)
    step(121, 121, live_i=True, live_g=True, live_w=True)
    step(122, 122, live_i=False, live_g=True, live_w=True)
    step(123, 123, live_i=False, live_g=False, live_w=False)
    step(124, 124, live_i=False, live_g=False, live_w=False)
    s_wait(124 % 3)
    plsc.subcore_barrier()

    pltpu.sync_copy(acc_sh.at[pl.ds(n0, COVER)],
                    part_hbm.at[cid, pl.ds(n0, COVER)])
    pltpu.sync_copy(rs_sh.at[pl.ds(n0, COVER)],
                    rsum_hbm.at[cid, pl.ds(n0, COVER)])


_sc_call = pl.kernel(
    _sc_body,
    out_type=(
        jax.ShapeDtypeStruct((NC, N, D), jnp.float32),
        jax.ShapeDtypeStruct((NC, N), jnp.float32),
    ),
    mesh=plsc.VectorSubcoreMesh(core_axis_name="c", subcore_axis_name="s",
                                num_cores=NC, num_subcores=NS),
    compiler_params=pltpu.CompilerParams(use_tc_tiling_on_sc=False,
                                         needs_layout_passes=False),
    scratch_types=[
        pltpu.VMEM((N,), jnp.int32),
        pltpu.VMEM((4, CHUNK), jnp.int32),
        pltpu.VMEM((3, CHUNK), jnp.int32),
        pltpu.VMEM((3, CHUNK), jnp.int32),
        pltpu.VMEM((3, CHUNK), jnp.float32),
        pltpu.VMEM((CHUNK, D), jnp.float32),
        pltpu.VMEM((CHUNK, D), jnp.float32),
        pltpu.VMEM((CHUNK, D), jnp.float32),
        pltpu.VMEM_SHARED((N, D), jnp.float32),
        pltpu.VMEM_SHARED((N,), jnp.float32),
        [pltpu.SemaphoreType.DMA] * 4,
        [pltpu.SemaphoreType.DMA] * 3,
        [pltpu.SemaphoreType.DMA] * 3,
    ],
)


def _comb_body(h1_ref, part_ref, rsum_ref, o_ref):
    acc = part_ref[0] + part_ref[1]
    rs = rsum_ref[0] + rsum_ref[1]
    denom = jnp.where(rs == 0.0, 1e-12, rs)
    h = (h1_ref[...] * rs + acc) / denom
    o_ref[...] = jnp.where(h > 0.0, h, jnp.exp(h) - 1.0)


_comb_call = pl.pallas_call(
    _comb_body,
    out_shape=jax.ShapeDtypeStruct((N, D), jnp.float32),
)


def _pack_s(s1, s2):
    lo = lax.bitcast_convert_type(s1.astype(jnp.bfloat16), jnp.uint16)
    hi = lax.bitcast_convert_type(s2.astype(jnp.bfloat16), jnp.uint16)
    word = lo.astype(jnp.uint32) | (hi.astype(jnp.uint32) << 16)
    return lax.bitcast_convert_type(word, jnp.int32)


def kernel(input_, edge, W, a):
    edge = edge.astype(jnp.int32)
    epk = edge[0] | (edge[1] << 16)
    h1, h2, s1, s2 = _mm_call(input_, W, a)
    st = _pack_s(s1.reshape(N), s2.reshape(N))
    part, rsum = _sc_call(epk, st, h2)
    return _comb_call(h1, part, rsum.reshape(NC, N, 1))
